# Initial kernel scaffold; baseline (speedup 1.0000x reference)
import jax
import jax.numpy as jnp
from jax.experimental import pallas as pl


def kernel(x, positions, edge_index, W, b):
    raise NotImplementedError("write your pallas kernel here")


# speedup vs baseline: 5.2780x; 5.2780x over previous
import functools

import jax
import jax.numpy as jnp
from jax import lax
from jax.experimental import pallas as pl
from jax.experimental.pallas import tpu as pltpu
from jax.experimental.pallas import tpu_sc as plsc


def _sc_accumulate(x144, pos16, src_p, dst_p, npad, epad):
  n_cores = 2
  n_sub = 16
  chunk = 128
  rows_per_tile = npad // n_sub
  strips = rows_per_tile // chunk
  edges_per_core = epad // n_cores
  edges_per_tile = edges_per_core // n_sub
  n_iters = edges_per_tile // chunk

  mesh = plsc.VectorSubcoreMesh(core_axis_name="c", subcore_axis_name="s")

  @functools.partial(
      pl.kernel,
      mesh=mesh,
      compiler_params=pltpu.CompilerParams(use_tc_tiling_on_sc=False),
      out_type=(
          jax.ShapeDtypeStruct((n_cores, npad, 144), jnp.float32),
          jax.ShapeDtypeStruct((n_cores, npad, 16), jnp.float32),
      ),
      scratch_types=[
          pltpu.VMEM((chunk,), jnp.int32),
          pltpu.VMEM((chunk,), jnp.int32),
          pltpu.VMEM((chunk, 144), jnp.float32),
          pltpu.VMEM((chunk, 16), jnp.float32),
          pltpu.SemaphoreType.DMA,
          pltpu.SemaphoreType.DMA,
          pltpu.VMEM_SHARED((npad, 144), jnp.float32),
          pltpu.VMEM_SHARED((npad, 16), jnp.float32),
      ],
  )
  def sc_kernel(x_hbm, p_hbm, src_hbm, dst_hbm, agg_out, ns_out,
                sidx, didx, xrows, prows, sem1, sem2, agg_sh, ns_sh):
    c = lax.axis_index("c")
    s = lax.axis_index("s")

    @pl.loop(0, chunk)
    def _(i):
      for j in range(9):
        xrows[i, pl.ds(j * 16, 16)] = jnp.zeros((16,), jnp.float32)
      prows[i, :] = jnp.zeros((16,), jnp.float32)

    @pl.loop(0, strips)
    def _(k):
      base = s * rows_per_tile + k * chunk
      pltpu.sync_copy(xrows, agg_sh.at[pl.ds(base, chunk)])
      pltpu.sync_copy(prows, ns_sh.at[pl.ds(base, chunk)])

    plsc.subcore_barrier()

    @pl.loop(0, n_iters)
    def _(i):
      ebase = c * edges_per_core + s * edges_per_tile + i * chunk
      pltpu.sync_copy(src_hbm.at[pl.ds(ebase, chunk)], sidx)
      pltpu.sync_copy(dst_hbm.at[pl.ds(ebase, chunk)], didx)
      cp1 = pltpu.async_copy(x_hbm.at[sidx], xrows, sem1)
      cp2 = pltpu.async_copy(p_hbm.at[didx], prows, sem2)
      cp1.wait()
      cp2.wait()
      pltpu.sync_copy(xrows, agg_sh.at[didx], add=True)
      pltpu.sync_copy(prows, ns_sh.at[sidx], add=True)

    plsc.subcore_barrier()

    base = s * rows_per_tile
    pltpu.sync_copy(agg_sh.at[pl.ds(base, rows_per_tile)],
                    agg_out.at[c, pl.ds(base, rows_per_tile)])
    pltpu.sync_copy(ns_sh.at[pl.ds(base, rows_per_tile)],
                    ns_out.at[c, pl.ds(base, rows_per_tile)])

  return sc_kernel(x144, pos16, src_p, dst_p)


def _tc_body(x_ref, p_ref, agg_ref, ns_ref, w_ref, b_ref, o_ref):
  agg2 = agg_ref[...]
  aggs = agg2[0] + agg2[1]
  deg = jnp.maximum(aggs[:, 128:129], 1.0)
  agg = aggs[:, 0:128] / deg
  ns2 = ns_ref[...]
  ns = ns2[0] + ns2[1]
  cnt = jnp.maximum(ns[:, 3:4], 1.0)
  d = p_ref[...] - ns / cnt
  col = lax.broadcasted_iota(jnp.int32, d.shape, 1)
  d = jnp.where(col < 3, d, 0.0)
  curv = jnp.sqrt(jnp.sum(d * d, axis=1, keepdims=True))
  h = (x_ref[...] + agg) * (1.0 + curv)
  o_ref[...] = (
      jnp.dot(h, w_ref[...], preferred_element_type=jnp.float32) + b_ref[...]
  )


def kernel(x, positions, edge_index, W, b):
  N, D = x.shape
  E = edge_index.shape[1]
  npad = ((N + 2047) // 2048) * 2048
  epad = ((E + 4095) // 4096) * 4096
  dummy = N

  ei = jnp.clip(edge_index.astype(jnp.int32), 0, N - 1)
  src = ei[0]
  dst = ei[1]
  pad = epad - E
  src_p = jnp.concatenate([src, jnp.zeros((pad,), jnp.int32)])
  dst_p = jnp.concatenate([dst, jnp.full((pad,), dummy, jnp.int32)])

  x144 = jnp.concatenate(
      [x, jnp.ones((N, 1), jnp.float32), jnp.zeros((N, 15), jnp.float32)],
      axis=1)
  pos16 = jnp.zeros((npad, 16), jnp.float32)
  pos16 = pos16.at[:N, 0:3].set(positions)
  pos16 = pos16.at[:N, 3].set(1.0)

  agg_out, ns_out = _sc_accumulate(x144, pos16, src_p, dst_p, npad, epad)

  blk = 1000
  grid = (N // blk,)
  out = pl.pallas_call(
      _tc_body,
      grid=grid,
      in_specs=[
          pl.BlockSpec((blk, D), lambda i: (i, 0)),
          pl.BlockSpec((blk, 16), lambda i: (i, 0)),
          pl.BlockSpec((2, blk, 144), lambda i: (0, i, 0)),
          pl.BlockSpec((2, blk, 16), lambda i: (0, i, 0)),
          pl.BlockSpec((D, D), lambda i: (0, 0)),
          pl.BlockSpec((1, D), lambda i: (0, 0)),
      ],
      out_specs=pl.BlockSpec((blk, D), lambda i: (i, 0)),
      out_shape=jax.ShapeDtypeStruct((N, D), jnp.float32),
  )(x, pos16, agg_out, ns_out, W, b.reshape(1, D))
  return out


# speedup vs baseline: 7.3010x; 1.3833x over previous
import functools

import jax
import jax.numpy as jnp
from jax import lax
from jax.experimental import pallas as pl
from jax.experimental.pallas import tpu as pltpu
from jax.experimental.pallas import tpu_sc as plsc


def _sc_accumulate(x144, pos16, src_p, dst_p, npad, epad, chunk, n_iters):
  n_cores = 2
  n_sub = 16
  rows_per_tile = npad // n_sub
  strips = rows_per_tile // chunk

  mesh = plsc.VectorSubcoreMesh(core_axis_name="c", subcore_axis_name="s")

  @functools.partial(
      pl.kernel,
      mesh=mesh,
      compiler_params=pltpu.CompilerParams(use_tc_tiling_on_sc=False),
      out_type=(
          jax.ShapeDtypeStruct((n_cores, npad, 144), jnp.float32),
          jax.ShapeDtypeStruct((n_cores, npad, 16), jnp.float32),
      ),
      scratch_types=[
          pltpu.VMEM((2, chunk), jnp.int32),
          pltpu.VMEM((2, chunk), jnp.int32),
          pltpu.VMEM((2, chunk, 144), jnp.float32),
          pltpu.VMEM((2, chunk, 16), jnp.float32),
          pltpu.SemaphoreType.DMA,
          pltpu.SemaphoreType.DMA,
          pltpu.SemaphoreType.DMA,
          pltpu.SemaphoreType.DMA,
          pltpu.VMEM_SHARED((npad, 144), jnp.float32),
          pltpu.VMEM_SHARED((npad, 16), jnp.float32),
      ],
  )
  def sc_kernel(x_hbm, p_hbm, src_hbm, dst_hbm, agg_out, ns_out,
                sidx, didx, xrows, prows, gsem0, gsem1, isem0, isem1,
                agg_sh, ns_sh):
    c = lax.axis_index("c")
    s = lax.axis_index("s")
    gsems = (gsem0, gsem1)
    isems = (isem0, isem1)

    @pl.loop(0, chunk)
    def _(i):
      for j in range(9):
        xrows[0, i, pl.ds(j * 16, 16)] = jnp.zeros((16,), jnp.float32)
      prows[0, i, :] = jnp.zeros((16,), jnp.float32)

    @pl.loop(0, strips)
    def _(k):
      base = s * rows_per_tile + k * chunk
      pltpu.sync_copy(xrows.at[0], agg_sh.at[pl.ds(base, chunk)])
      pltpu.sync_copy(prows.at[0], ns_sh.at[pl.ds(base, chunk)])

    plsc.subcore_barrier()

    row0 = (c * n_sub + s) * n_iters

    def idx_pair(i, b):
      return (
          (src_hbm.at[row0 + i], sidx.at[b], isems[b]),
          (dst_hbm.at[row0 + i], didx.at[b], isems[b]),
      )

    def gather_pair(i, b):
      del i
      return (
          (x_hbm.at[sidx.at[b]], xrows.at[b], gsems[b]),
          (p_hbm.at[didx.at[b]], prows.at[b], gsems[b]),
      )

    def issue(pair):
      for args in pair:
        pltpu.async_copy(*args)

    def wait(pair):
      for args in pair:
        pltpu.make_async_copy(*args).wait()

    def do_scatters(i, b):
      del i
      pltpu.sync_copy(xrows.at[b], agg_sh.at[didx.at[b]], add=True)
      pltpu.sync_copy(prows.at[b], ns_sh.at[sidx.at[b]], add=True)

    issue(idx_pair(0, 0))
    wait(idx_pair(0, 0))
    issue(idx_pair(1, 1))
    issue(gather_pair(0, 0))

    @pl.loop(0, n_iters, step=2)
    def _(i0):
      for b in range(2):
        i = i0 + b
        wait(gather_pair(i, b))

        @pl.when(i + 1 < n_iters)
        def _():
          wait(idx_pair(i + 1, 1 - b))
          issue(gather_pair(i + 1, 1 - b))

        do_scatters(i, b)

        @pl.when(i + 2 < n_iters)
        def _():
          issue(idx_pair(i + 2, b))

    plsc.subcore_barrier()

    base = s * rows_per_tile
    pltpu.sync_copy(agg_sh.at[pl.ds(base, rows_per_tile)],
                    agg_out.at[c, pl.ds(base, rows_per_tile)])
    pltpu.sync_copy(ns_sh.at[pl.ds(base, rows_per_tile)],
                    ns_out.at[c, pl.ds(base, rows_per_tile)])

  return sc_kernel(x144, pos16, src_p, dst_p)


def _tc_body(x_ref, p_ref, agg_ref, ns_ref, w_ref, b_ref, o_ref):
  agg2 = agg_ref[...]
  aggs = agg2[0] + agg2[1]
  deg = jnp.maximum(aggs[:, 128:129], 1.0)
  agg = aggs[:, 0:128] / deg
  ns2 = ns_ref[...]
  ns = ns2[0] + ns2[1]
  cnt = jnp.maximum(ns[:, 3:4], 1.0)
  d = p_ref[...] - ns / cnt
  col = lax.broadcasted_iota(jnp.int32, d.shape, 1)
  d = jnp.where(col < 3, d, 0.0)
  curv = jnp.sqrt(jnp.sum(d * d, axis=1, keepdims=True))
  h = (x_ref[...] + agg) * (1.0 + curv)
  o_ref[...] = (
      jnp.dot(h, w_ref[...], preferred_element_type=jnp.float32) + b_ref[...]
  )


def kernel(x, positions, edge_index, W, b):
  N, D = x.shape
  E = edge_index.shape[1]
  chunk = 80
  n_iters = -(-E // (32 * chunk))
  n_iters += n_iters % 2
  epad = 32 * chunk * n_iters
  npad = ((N + 16 * chunk - 1) // (16 * chunk)) * (16 * chunk)
  dummy = N

  ei = jnp.clip(edge_index.astype(jnp.int32), 0, N - 1)
  src = ei[0]
  dst = ei[1]
  pad = epad - E
  src_p = jnp.concatenate([src, jnp.zeros((pad,), jnp.int32)]
                          ).reshape(-1, chunk)
  dst_p = jnp.concatenate([dst, jnp.full((pad,), dummy, jnp.int32)]
                          ).reshape(-1, chunk)

  x144 = jnp.concatenate(
      [x, jnp.ones((N, 1), jnp.float32), jnp.zeros((N, 15), jnp.float32)],
      axis=1)
  pos16 = jnp.zeros((npad, 16), jnp.float32)
  pos16 = pos16.at[:N, 0:3].set(positions)
  pos16 = pos16.at[:N, 3].set(1.0)

  agg_out, ns_out = _sc_accumulate(x144, pos16, src_p, dst_p, npad, epad,
                                   chunk, n_iters)

  blk = 1000
  grid = (N // blk,)
  out = pl.pallas_call(
      _tc_body,
      grid=grid,
      in_specs=[
          pl.BlockSpec((blk, D), lambda i: (i, 0)),
          pl.BlockSpec((blk, 16), lambda i: (i, 0)),
          pl.BlockSpec((2, blk, 144), lambda i: (0, i, 0)),
          pl.BlockSpec((2, blk, 16), lambda i: (0, i, 0)),
          pl.BlockSpec((D, D), lambda i: (0, 0)),
          pl.BlockSpec((1, D), lambda i: (0, 0)),
      ],
      out_specs=pl.BlockSpec((blk, D), lambda i: (i, 0)),
      out_shape=jax.ShapeDtypeStruct((N, D), jnp.float32),
  )(x, pos16, agg_out, ns_out, W, b.reshape(1, D))
  return out


# speedup vs baseline: 10.0946x; 1.3826x over previous
import functools

import jax
import jax.numpy as jnp
from jax import lax
from jax.experimental import pallas as pl
from jax.experimental.pallas import tpu as pltpu
from jax.experimental.pallas import tpu_sc as plsc


def _sc_accumulate(x, pos16, src_p, dst_p, npad, n0, n1, chunk):
  n_cores = 2
  n_sub = 16
  d_feat = x.shape[1]
  rows_per_tile = npad // n_sub
  strips = rows_per_tile // chunk

  mesh = plsc.VectorSubcoreMesh(core_axis_name="c", subcore_axis_name="s")

  @functools.partial(
      pl.kernel,
      mesh=mesh,
      compiler_params=pltpu.CompilerParams(use_tc_tiling_on_sc=False),
      out_type=(
          jax.ShapeDtypeStruct((n_cores, npad, d_feat), jnp.float32),
          jax.ShapeDtypeStruct((n_cores, npad, 16), jnp.float32),
          jax.ShapeDtypeStruct((n_cores, npad, 16), jnp.float32),
      ),
      scratch_types=[
          pltpu.VMEM((2, chunk), jnp.int32),
          pltpu.VMEM((2, chunk), jnp.int32),
          pltpu.VMEM((2, chunk, d_feat), jnp.float32),
          pltpu.VMEM((2, chunk, 16), jnp.float32),
          pltpu.VMEM((2, chunk, 16), jnp.float32),
          pltpu.SemaphoreType.DMA,
          pltpu.SemaphoreType.DMA,
          pltpu.SemaphoreType.DMA,
          pltpu.SemaphoreType.DMA,
          pltpu.VMEM_SHARED((npad, d_feat), jnp.float32),
          pltpu.VMEM_SHARED((npad, 16), jnp.float32),
          pltpu.VMEM_SHARED((npad, 16), jnp.float32),
      ],
  )
  def sc_kernel(x_hbm, p_hbm, src_hbm, dst_hbm, agg_out, ns_out, dg_out,
                sidx, didx, xrows, prows, qrows, gsem0, gsem1, isem0, isem1,
                agg_sh, ns_sh, dg_sh):
    c = lax.axis_index("c")
    s = lax.axis_index("s")
    gsems = (gsem0, gsem1)
    isems = (isem0, isem1)

    @pl.loop(0, chunk)
    def _(i):
      for j in range(d_feat // 16):
        xrows[0, i, pl.ds(j * 16, 16)] = jnp.zeros((16,), jnp.float32)
      prows[0, i, :] = jnp.zeros((16,), jnp.float32)
      qrows[0, i, :] = jnp.zeros((16,), jnp.float32)

    @pl.loop(0, strips)
    def _(k):
      base = s * rows_per_tile + k * chunk
      pltpu.sync_copy(xrows.at[0], agg_sh.at[pl.ds(base, chunk)])
      pltpu.sync_copy(prows.at[0], ns_sh.at[pl.ds(base, chunk)])
      pltpu.sync_copy(qrows.at[0], dg_sh.at[pl.ds(base, chunk)])

    plsc.subcore_barrier()

    def idx_pair(i, b, row0):
      return (
          (src_hbm.at[row0 + i], sidx.at[b], isems[b]),
          (dst_hbm.at[row0 + i], didx.at[b], isems[b]),
      )

    def gather_trip(b):
      return (
          (x_hbm.at[sidx.at[b]], xrows.at[b], gsems[b]),
          (p_hbm.at[didx.at[b]], prows.at[b], gsems[b]),
          (p_hbm.at[sidx.at[b]], qrows.at[b], gsems[b]),
      )

    def issue(pairs):
      for args in pairs:
        pltpu.async_copy(*args)

    def wait(pairs):
      for args in pairs:
        pltpu.make_async_copy(*args).wait()

    def do_scatters(b):
      pltpu.sync_copy(xrows.at[b], agg_sh.at[didx.at[b]], add=True)
      pltpu.sync_copy(prows.at[b], ns_sh.at[sidx.at[b]], add=True)
      pltpu.sync_copy(qrows.at[b], dg_sh.at[didx.at[b]], add=True)

    def pipeline(n_it, row0):
      issue(idx_pair(0, 0, row0))
      wait(idx_pair(0, 0, row0))
      issue(idx_pair(1, 1, row0))
      issue(gather_trip(0))

      n_even = n_it - (n_it % 2)

      @pl.loop(0, n_even, step=2)
      def _(i0):
        for b in range(2):
          i = i0 + b
          wait(gather_trip(b))

          @pl.when(i + 1 < n_it)
          def _():
            wait(idx_pair(i + 1, 1 - b, row0))
            issue(gather_trip(1 - b))

          do_scatters(b)

          @pl.when(i + 2 < n_it)
          def _():
            issue(idx_pair(i + 2, b, row0))

      if n_it % 2:
        wait(gather_trip((n_it - 1) % 2))
        do_scatters((n_it - 1) % 2)

    @pl.when(c == 0)
    def _():
      pipeline(n0, s * n0)

    @pl.when(c == 1)
    def _():
      pipeline(n1, n_sub * n0 + s * n1)

    plsc.subcore_barrier()

    base = s * rows_per_tile
    pltpu.sync_copy(agg_sh.at[pl.ds(base, rows_per_tile)],
                    agg_out.at[c, pl.ds(base, rows_per_tile)])
    pltpu.sync_copy(ns_sh.at[pl.ds(base, rows_per_tile)],
                    ns_out.at[c, pl.ds(base, rows_per_tile)])
    pltpu.sync_copy(dg_sh.at[pl.ds(base, rows_per_tile)],
                    dg_out.at[c, pl.ds(base, rows_per_tile)])

  return sc_kernel(x, pos16, src_p, dst_p)


def _tc_body(x_ref, p_ref, agg_ref, ns_ref, dg_ref, w_ref, b_ref, o_ref):
  agg2 = agg_ref[...]
  agg = agg2[0] + agg2[1]
  dg2 = dg_ref[...]
  deg = jnp.maximum(dg2[0, :, 3:4] + dg2[1, :, 3:4], 1.0)
  ns2 = ns_ref[...]
  ns = ns2[0] + ns2[1]
  cnt = jnp.maximum(ns[:, 3:4], 1.0)
  d = p_ref[...] - ns / cnt
  col = lax.broadcasted_iota(jnp.int32, d.shape, 1)
  d = jnp.where(col < 3, d, 0.0)
  curv = jnp.sqrt(jnp.sum(d * d, axis=1, keepdims=True))
  h = (x_ref[...] + agg / deg) * (1.0 + curv)
  o_ref[...] = (
      jnp.dot(h, w_ref[...], preferred_element_type=jnp.float32) + b_ref[...]
  )


def kernel(x, positions, edge_index, W, b):
  N, D = x.shape
  E = edge_index.shape[1]
  chunk = 80
  n_sub = 16
  rows_total = -(-E // chunk)
  t_per_tile = -(-rows_total // n_sub)
  n0 = max(2, round(t_per_tile * 0.6187))
  n1 = max(2, t_per_tile - n0)
  epad = n_sub * (n0 + n1) * chunk
  npad = ((N + n_sub * chunk - 1) // (n_sub * chunk)) * (n_sub * chunk)
  dummy = N

  src = edge_index[0].astype(jnp.int32)
  dst = edge_index[1].astype(jnp.int32)
  pad = epad - E
  if pad:
    src = jnp.concatenate([src, jnp.zeros((pad,), jnp.int32)])
    dst = jnp.concatenate([dst, jnp.full((pad,), dummy, jnp.int32)])
  src_p = src.reshape(-1, chunk)
  dst_p = dst.reshape(-1, chunk)

  pos16 = jnp.zeros((npad, 16), jnp.float32)
  pos16 = pos16.at[:N, 0:3].set(positions)
  pos16 = pos16.at[:N, 3].set(1.0)

  agg_out, ns_out, dg_out = _sc_accumulate(x, pos16, src_p, dst_p, npad,
                                           n0, n1, chunk)

  blk = 1000
  grid = (N // blk,)
  out = pl.pallas_call(
      _tc_body,
      grid=grid,
      in_specs=[
          pl.BlockSpec((blk, D), lambda i: (i, 0)),
          pl.BlockSpec((blk, 16), lambda i: (i, 0)),
          pl.BlockSpec((2, blk, D), lambda i: (0, i, 0)),
          pl.BlockSpec((2, blk, 16), lambda i: (0, i, 0)),
          pl.BlockSpec((2, blk, 16), lambda i: (0, i, 0)),
          pl.BlockSpec((D, D), lambda i: (0, 0)),
          pl.BlockSpec((1, D), lambda i: (0, 0)),
      ],
      out_specs=pl.BlockSpec((blk, D), lambda i: (i, 0)),
      out_shape=jax.ShapeDtypeStruct((N, D), jnp.float32),
  )(x, pos16, agg_out, ns_out, dg_out, W, b.reshape(1, D))
  return out


# speedup vs baseline: 13.7149x; 1.3586x over previous
import functools

import jax
import jax.numpy as jnp
from jax import lax
from jax.experimental import pallas as pl
from jax.experimental.pallas import tpu as pltpu
from jax.experimental.pallas import tpu_sc as plsc


def _sc_accumulate(x, pos16, ei, npad, n0, n1, chunk):
  n_cores = 2
  n_sub = 16
  d_feat = x.shape[1]
  rows_per_tile = npad // n_sub
  strips = rows_per_tile // chunk

  mesh = plsc.VectorSubcoreMesh(core_axis_name="c", subcore_axis_name="s")

  @functools.partial(
      pl.kernel,
      mesh=mesh,
      compiler_params=pltpu.CompilerParams(use_tc_tiling_on_sc=False),
      out_type=(
          jax.ShapeDtypeStruct((n_cores, npad, d_feat), jnp.float32),
          jax.ShapeDtypeStruct((n_cores, npad, 16), jnp.float32),
          jax.ShapeDtypeStruct((n_cores, npad, 16), jnp.float32),
      ),
      scratch_types=[
          pltpu.VMEM((2, chunk), jnp.int32),
          pltpu.VMEM((2, chunk), jnp.int32),
          pltpu.VMEM((2, chunk, d_feat), jnp.float32),
          pltpu.VMEM((2, chunk, 16), jnp.float32),
          pltpu.VMEM((chunk, 16), jnp.float32),
          pltpu.SemaphoreType.DMA,
          pltpu.SemaphoreType.DMA,
          pltpu.SemaphoreType.DMA,
          pltpu.SemaphoreType.DMA,
          pltpu.VMEM_SHARED((npad, d_feat), jnp.float32),
          pltpu.VMEM_SHARED((npad, 16), jnp.float32),
          pltpu.VMEM_SHARED((npad, 16), jnp.float32),
      ],
  )
  def sc_kernel(x_hbm, p_hbm, ei_hbm, agg_out, ns_out, dg_out,
                sidx, didx, xrows, prows, ones16, gsem0, gsem1, isem0, isem1,
                agg_sh, ns_sh, dg_sh):
    c = lax.axis_index("c")
    s = lax.axis_index("s")
    gsems = (gsem0, gsem1)
    isems = (isem0, isem1)

    onehot3 = jnp.where(lax.iota(jnp.int32, 16) == 3, 1.0, 0.0)

    @pl.loop(0, chunk)
    def _(i):
      for j in range(d_feat // 16):
        xrows[0, i, pl.ds(j * 16, 16)] = jnp.zeros((16,), jnp.float32)
      prows[0, i, :] = jnp.zeros((16,), jnp.float32)
      ones16[i, :] = onehot3

    @pl.loop(0, strips)
    def _(k):
      base = s * rows_per_tile + k * chunk
      pltpu.sync_copy(xrows.at[0], agg_sh.at[pl.ds(base, chunk)])
      pltpu.sync_copy(prows.at[0], ns_sh.at[pl.ds(base, chunk)])
      pltpu.sync_copy(prows.at[0], dg_sh.at[pl.ds(base, chunk)])

    plsc.subcore_barrier()

    def idx_pair(i, b, row0):
      return (
          (ei_hbm.at[0, pl.ds((row0 + i) * chunk, chunk)], sidx.at[b],
           isems[b]),
          (ei_hbm.at[1, pl.ds((row0 + i) * chunk, chunk)], didx.at[b],
           isems[b]),
      )

    def gather_pair(b):
      return (
          (x_hbm.at[sidx.at[b]], xrows.at[b], gsems[b]),
          (p_hbm.at[didx.at[b]], prows.at[b], gsems[b]),
      )

    def issue(pairs):
      for args in pairs:
        pltpu.async_copy(*args)

    def wait(pairs):
      for args in pairs:
        pltpu.make_async_copy(*args).wait()

    def do_scatters(b):
      pltpu.sync_copy(xrows.at[b], agg_sh.at[didx.at[b]], add=True)
      pltpu.sync_copy(prows.at[b], ns_sh.at[sidx.at[b]], add=True)
      pltpu.sync_copy(ones16, dg_sh.at[didx.at[b]], add=True)

    def pipeline(n_it, row0):
      issue(idx_pair(0, 0, row0))
      wait(idx_pair(0, 0, row0))
      issue(idx_pair(1, 1, row0))
      issue(gather_pair(0))

      n_even = n_it - (n_it % 2)

      @pl.loop(0, n_even, step=2)
      def _(i0):
        for b in range(2):
          i = i0 + b
          wait(gather_pair(b))

          @pl.when(i + 1 < n_it)
          def _():
            wait(idx_pair(i + 1, 1 - b, row0))
            issue(gather_pair(1 - b))

          do_scatters(b)

          @pl.when(i + 2 < n_it)
          def _():
            issue(idx_pair(i + 2, b, row0))

      if n_it % 2:
        wait(gather_pair((n_it - 1) % 2))
        do_scatters((n_it - 1) % 2)

    @pl.when(c == 0)
    def _():
      pipeline(n0, s * n0)

    @pl.when(c == 1)
    def _():
      pipeline(n1, n_sub * n0 + s * n1)

    plsc.subcore_barrier()

    base = s * rows_per_tile
    pltpu.sync_copy(agg_sh.at[pl.ds(base, rows_per_tile)],
                    agg_out.at[c, pl.ds(base, rows_per_tile)])
    pltpu.sync_copy(ns_sh.at[pl.ds(base, rows_per_tile)],
                    ns_out.at[c, pl.ds(base, rows_per_tile)])
    pltpu.sync_copy(dg_sh.at[pl.ds(base, rows_per_tile)],
                    dg_out.at[c, pl.ds(base, rows_per_tile)])

  return sc_kernel(x, pos16, ei)


def _tc_body(x_ref, p_ref, agg_ref, ns_ref, dg_ref, w_ref, b_ref, o_ref):
  agg2 = agg_ref[...]
  agg = agg2[0] + agg2[1]
  dg2 = dg_ref[...]
  deg = jnp.maximum(dg2[0, :, 3:4] + dg2[1, :, 3:4], 1.0)
  ns2 = ns_ref[...]
  ns = ns2[0] + ns2[1]
  cnt = jnp.maximum(ns[:, 3:4], 1.0)
  d = p_ref[...] - ns / cnt
  col = lax.broadcasted_iota(jnp.int32, d.shape, 1)
  d = jnp.where(col < 3, d, 0.0)
  curv = jnp.sqrt(jnp.sum(d * d, axis=1, keepdims=True))
  h = (x_ref[...] + agg / deg) * (1.0 + curv)
  o_ref[...] = (
      jnp.dot(h, w_ref[...], preferred_element_type=jnp.float32) + b_ref[...]
  )


def kernel(x, positions, edge_index, W, b):
  N, D = x.shape
  E = edge_index.shape[1]
  chunk = 80
  n_sub = 16
  rows_total = -(-E // chunk)
  t_per_tile = -(-rows_total // n_sub)
  n0 = max(2, round(t_per_tile * 0.515))
  n1 = max(2, t_per_tile - n0)
  epad = n_sub * (n0 + n1) * chunk
  npad = ((N + n_sub * chunk - 1) // (n_sub * chunk)) * (n_sub * chunk)
  dummy = N

  ei = jnp.asarray(edge_index, jnp.int32)
  pad = epad - E
  if pad:
    pad_block = jnp.concatenate(
        [jnp.zeros((1, pad), jnp.int32), jnp.full((1, pad), dummy, jnp.int32)])
    ei = jnp.concatenate([ei, pad_block], axis=1)

  pos16 = jnp.concatenate(
      [positions, jnp.ones((N, 1), jnp.float32),
       jnp.zeros((N, 12), jnp.float32)], axis=1)
  pos16 = jnp.concatenate(
      [pos16, jnp.zeros((npad - N, 16), jnp.float32)])

  agg_out, ns_out, dg_out = _sc_accumulate(x, pos16, ei, npad, n0, n1, chunk)

  blk = 1000
  grid = (N // blk,)
  out = pl.pallas_call(
      _tc_body,
      grid=grid,
      in_specs=[
          pl.BlockSpec((blk, D), lambda i: (i, 0)),
          pl.BlockSpec((blk, 16), lambda i: (i, 0)),
          pl.BlockSpec((2, blk, D), lambda i: (0, i, 0)),
          pl.BlockSpec((2, blk, 16), lambda i: (0, i, 0)),
          pl.BlockSpec((2, blk, 16), lambda i: (0, i, 0)),
          pl.BlockSpec((D, D), lambda i: (0, 0)),
          pl.BlockSpec((1, D), lambda i: (0, 0)),
      ],
      out_specs=pl.BlockSpec((blk, D), lambda i: (i, 0)),
      out_shape=jax.ShapeDtypeStruct((N, D), jnp.float32),
  )(x, pos16, agg_out, ns_out, dg_out, W, b.reshape(1, D))
  return out
